# Initial kernel scaffold; baseline (speedup 1.0000x reference)
import jax
import jax.numpy as jnp
from jax.experimental import pallas as pl


def kernel(x, edge_index, adj1_values, adj2_values, node_index, kernel_f, kernel_mean, kernel_var):
    raise NotImplementedError("write your pallas kernel here")


# speedup vs baseline: 5.1431x; 5.1431x over previous
import functools

import jax
import jax.numpy as jnp
from jax import lax
from jax.experimental import pallas as pl
from jax.experimental.pallas import tpu as pltpu
from jax.experimental.pallas import tpu_sc as plsc

N = 10000
E = 320000
D_IN = 128
D_HID = 64
D_OUT = 7
B = 1000
GAMMA = 1.0

NP = 10240
CH = 128
EPS_W = 20096
EP = EPS_W * 16
NCHUNK = EPS_W // CH
RSUB = NP // 16
BP = 1024
NW = 32

_mesh = plsc.VectorSubcoreMesh(
    core_axis_name="c", subcore_axis_name="s", num_cores=2, num_subcores=16)

_sc_params = pltpu.CompilerParams(
    needs_layout_passes=False, use_tc_tiling_on_sc=False)


def _tca_body(x_ref, kf_ref, om_ref, ov_ref):
    h = jnp.dot(x_ref[...], kf_ref[...], preferred_element_type=jnp.float32)
    m = jnp.maximum(h, 0.0)
    a = jnp.exp(-GAMMA * m)
    ma = m * a
    om_ref[...] = ma
    ov_ref[...] = ma * a


def _tc_a(xp, kf):
    return pl.pallas_call(
        _tca_body,
        grid=(NP // 256,),
        in_specs=[
            pl.BlockSpec((256, D_IN), lambda i: (i, 0)),
            pl.BlockSpec((D_IN, D_HID), lambda i: (0, 0)),
        ],
        out_specs=[pl.BlockSpec((256, D_HID), lambda i: (i, 0)),
                   pl.BlockSpec((256, D_HID), lambda i: (i, 0))],
        out_shape=[jax.ShapeDtypeStruct((NP, D_HID), jnp.float32),
                   jax.ShapeDtypeStruct((NP, D_HID), jnp.float32)],
    )(xp, kf)


def _tcb_body(am_ref, av_ref, km_ref, kv_ref, om_ref, ov_ref):
    m2 = jnp.dot(am_ref[...], km_ref[...], preferred_element_type=jnp.float32)
    v2 = jnp.maximum(
        jnp.dot(av_ref[...], kv_ref[...], preferred_element_type=jnp.float32),
        0.0)
    a = jnp.exp(-GAMMA * v2)
    om_ref[...] = m2 * a
    ov_ref[...] = v2 * a * a


def _tc_b(am, av, kmp, kvp):
    return pl.pallas_call(
        _tcb_body,
        grid=(NP // 256,),
        in_specs=[
            pl.BlockSpec((256, D_HID), lambda i: (i, 0)),
            pl.BlockSpec((256, D_HID), lambda i: (i, 0)),
            pl.BlockSpec((D_HID, 16), lambda i: (0, 0)),
            pl.BlockSpec((D_HID, 16), lambda i: (0, 0)),
        ],
        out_specs=[pl.BlockSpec((256, 16), lambda i: (i, 0)),
                   pl.BlockSpec((256, 16), lambda i: (i, 0))],
        out_shape=[jax.ShapeDtypeStruct((NP, 16), jnp.float32),
                   jax.ShapeDtypeStruct((NP, 16), jnp.float32)],
    )(am, av, kmp, kvp)


def _tcd_body(qm_ref, qv_ref, eps_ref, o_ref):
    o_ref[...] = qm_ref[...] + jnp.sqrt(qv_ref[...] + 1e-8) * eps_ref[...]


def _tc_d(qm, qv, epsp):
    return pl.pallas_call(
        _tcd_body,
        grid=(NP // 1024,),
        in_specs=[
            pl.BlockSpec((1024, 16), lambda i: (i, 0)),
            pl.BlockSpec((1024, 16), lambda i: (i, 0)),
            pl.BlockSpec((1024, 16), lambda i: (i, 0)),
        ],
        out_specs=pl.BlockSpec((1024, 16), lambda i: (i, 0)),
        out_shape=jax.ShapeDtypeStruct((NP, 16), jnp.float32),
    )(qm, qv, epsp)


def _make_spmm(d):
    n_vreg = d // 16

    @functools.partial(
        pl.kernel,
        out_type=[jax.ShapeDtypeStruct((NP, d), jnp.float32),
                  jax.ShapeDtypeStruct((NP, d), jnp.float32)],
        mesh=_mesh,
        compiler_params=_sc_params,
        scratch_types=[
            pltpu.VMEM((CH,), jnp.int32),
            pltpu.VMEM((CH,), jnp.int32),
            pltpu.VMEM((CH,), jnp.float32),
            pltpu.VMEM((CH, d), jnp.float32),
            pltpu.VMEM_SHARED((NP, d), jnp.float32),
            pltpu.SemaphoreType.DMA,
        ],
    )
    def spmm(msg_m, msg_v, src_hbm, dst_hbm, w1_hbm, w2_hbm, out_m, out_v,
             src_v, dst_v, w_v, rows_v, acc_sh, sem):
        cid = lax.axis_index("c")
        sid = lax.axis_index("s")

        zval = jnp.zeros((16,), jnp.float32)

        def zbody(i, _):
            for k in range(n_vreg):
                rows_v[i, pl.ds(k * 16, 16)] = zval
            return 0

        lax.fori_loop(0, CH, zbody, 0)
        for z in range(RSUB // CH):
            pltpu.sync_copy(rows_v, acc_sh.at[pl.ds(sid * RSUB + z * CH, CH)])
        plsc.subcore_barrier()

        base = sid * EPS_W

        def chunk_body(ci, _):
            off = base + ci * CH
            pltpu.sync_copy(src_hbm.at[pl.ds(off, CH)], src_v)
            pltpu.sync_copy(dst_hbm.at[pl.ds(off, CH)], dst_v)

            @pl.when(cid == 0)
            def _():
                pltpu.sync_copy(w1_hbm.at[pl.ds(off, CH)], w_v)
                pltpu.async_copy(msg_m.at[src_v], rows_v, sem).wait()

            @pl.when(cid == 1)
            def _():
                pltpu.sync_copy(w2_hbm.at[pl.ds(off, CH)], w_v)
                pltpu.async_copy(msg_v.at[src_v], rows_v, sem).wait()

            def escale(t, _):
                for u in range(8):
                    e = t * 8 + u
                    idx = jnp.full((16,), e, jnp.int32)
                    ws = plsc.load_gather(w_v, [idx])
                    for k in range(n_vreg):
                        rows_v[e, pl.ds(k * 16, 16)] = (
                            rows_v[e, pl.ds(k * 16, 16)] * ws)
                return 0

            lax.fori_loop(0, CH // 8, escale, 0)
            pltpu.sync_copy(rows_v, acc_sh.at[dst_v], add=True)
            return 0

        lax.fori_loop(0, NCHUNK, chunk_body, 0)
        plsc.subcore_barrier()

        sl = pl.ds(sid * RSUB, RSUB)

        @pl.when(cid == 0)
        def _():
            pltpu.sync_copy(acc_sh.at[sl], out_m.at[sl])

        @pl.when(cid == 1)
        def _():
            pltpu.sync_copy(acc_sh.at[sl], out_v.at[sl])

    return spmm


_spmm1 = _make_spmm(D_HID)
_spmm2 = _make_spmm(16)


@functools.partial(
    pl.kernel,
    out_type=jax.ShapeDtypeStruct((BP, 16), jnp.float32),
    mesh=_mesh,
    compiler_params=_sc_params,
    scratch_types=[
        pltpu.VMEM((BP // NW,), jnp.int32),
        pltpu.VMEM((BP // NW, 16), jnp.float32),
        pltpu.SemaphoreType.DMA,
    ],
)
def _gather_out(hs_hbm, idx_hbm, out_hbm, idx_v, rows_v, sem):
    wid = lax.axis_index("s") * 2 + lax.axis_index("c")
    b_per_w = BP // NW
    base = wid * b_per_w
    pltpu.sync_copy(idx_hbm.at[pl.ds(base, b_per_w)], idx_v)
    pltpu.async_copy(hs_hbm.at[idx_v], rows_v, sem).wait()
    pltpu.sync_copy(rows_v, out_hbm.at[pl.ds(base, b_per_w)])


def kernel(x, edge_index, adj1_values, adj2_values, node_index,
           kernel_f, kernel_mean, kernel_var):
    src = edge_index[0]
    dst = edge_index[1]
    epad = EP - E
    src_p = jnp.concatenate([src, jnp.zeros((epad,), jnp.int32)])
    dst_p = jnp.concatenate([dst, jnp.full((epad,), N, jnp.int32)])
    w1_p = jnp.concatenate([adj1_values, jnp.zeros((epad,), jnp.float32)])
    w2_p = jnp.concatenate([adj2_values, jnp.zeros((epad,), jnp.float32)])
    xp = jnp.pad(x, ((0, NP - N), (0, 0)))
    kmp = jnp.pad(kernel_mean, ((0, 0), (0, 16 - D_OUT)))
    kvp = jnp.pad(kernel_var, ((0, 0), (0, 16 - D_OUT)))
    eps = jax.random.normal(jax.random.key(42), (N, D_OUT), dtype=jnp.float32)
    epsp = jnp.pad(eps, ((0, NP - N), (0, 16 - D_OUT)))
    ni_p = jnp.pad(node_index, (0, BP - B))

    msg_m, msg_v = _tc_a(xp, kernel_f)
    am, av = _spmm1(msg_m, msg_v, src_p, dst_p, w1_p, w2_p)
    m2, v2 = _tc_b(am, av, kmp, kvp)
    qm, qv = _spmm2(m2, v2, src_p, dst_p, w1_p, w2_p)
    hs = _tc_d(qm, qv, epsp)
    out = _gather_out(hs, ni_p)
    return out[:B, :D_OUT]


# speedup vs baseline: 7.0244x; 1.3658x over previous
import functools

import jax
import jax.numpy as jnp
from jax import lax
from jax.experimental import pallas as pl
from jax.experimental.pallas import tpu as pltpu
from jax.experimental.pallas import tpu_sc as plsc

N = 10000
E = 320000
D_IN = 128
D_HID = 64
D_OUT = 7
B = 1000
GAMMA = 1.0

NP = 10240
EPS_W = 20480
EP = EPS_W * 16
RSUB = NP // 16
BP = 1024
NW = 32

_mesh = plsc.VectorSubcoreMesh(
    core_axis_name="c", subcore_axis_name="s", num_cores=2, num_subcores=16)

_sc_params = pltpu.CompilerParams(
    needs_layout_passes=False, use_tc_tiling_on_sc=False)


def _tca_body(x_ref, kf_ref, om_ref, ov_ref):
    h = jnp.dot(x_ref[...], kf_ref[...], preferred_element_type=jnp.float32)
    m = jnp.maximum(h, 0.0)
    a = jnp.exp(-GAMMA * m)
    ma = m * a
    om_ref[...] = ma
    ov_ref[...] = ma * a


def _tc_a(xp, kf):
    return pl.pallas_call(
        _tca_body,
        grid=(NP // 256,),
        in_specs=[
            pl.BlockSpec((256, D_IN), lambda i: (i, 0)),
            pl.BlockSpec((D_IN, D_HID), lambda i: (0, 0)),
        ],
        out_specs=[pl.BlockSpec((256, D_HID), lambda i: (i, 0)),
                   pl.BlockSpec((256, D_HID), lambda i: (i, 0))],
        out_shape=[jax.ShapeDtypeStruct((NP, D_HID), jnp.float32),
                   jax.ShapeDtypeStruct((NP, D_HID), jnp.float32)],
    )(xp, kf)


def _tcb_body(am_ref, av_ref, km_ref, kv_ref, om_ref, ov_ref):
    m2 = jnp.dot(am_ref[...], km_ref[...], preferred_element_type=jnp.float32)
    v2 = jnp.maximum(
        jnp.dot(av_ref[...], kv_ref[...], preferred_element_type=jnp.float32),
        0.0)
    a = jnp.exp(-GAMMA * v2)
    om_ref[...] = m2 * a
    ov_ref[...] = v2 * a * a


def _tc_b(am, av, kmp, kvp):
    return pl.pallas_call(
        _tcb_body,
        grid=(NP // 256,),
        in_specs=[
            pl.BlockSpec((256, D_HID), lambda i: (i, 0)),
            pl.BlockSpec((256, D_HID), lambda i: (i, 0)),
            pl.BlockSpec((D_HID, 16), lambda i: (0, 0)),
            pl.BlockSpec((D_HID, 16), lambda i: (0, 0)),
        ],
        out_specs=[pl.BlockSpec((256, 16), lambda i: (i, 0)),
                   pl.BlockSpec((256, 16), lambda i: (i, 0))],
        out_shape=[jax.ShapeDtypeStruct((NP, 16), jnp.float32),
                   jax.ShapeDtypeStruct((NP, 16), jnp.float32)],
    )(am, av, kmp, kvp)


def _tcd_body(qm_ref, qv_ref, eps_ref, o_ref):
    o_ref[...] = qm_ref[...] + jnp.sqrt(qv_ref[...] + 1e-8) * eps_ref[...]


def _tc_d(qm, qv, epsp):
    return pl.pallas_call(
        _tcd_body,
        grid=(NP // 1024,),
        in_specs=[
            pl.BlockSpec((1024, 16), lambda i: (i, 0)),
            pl.BlockSpec((1024, 16), lambda i: (i, 0)),
            pl.BlockSpec((1024, 16), lambda i: (i, 0)),
        ],
        out_specs=pl.BlockSpec((1024, 16), lambda i: (i, 0)),
        out_shape=jax.ShapeDtypeStruct((NP, 16), jnp.float32),
    )(qm, qv, epsp)


def _lane_splat(vec, u):
    return lax.gather(
        vec, jnp.full((16, 1), u, jnp.int32),
        lax.GatherDimensionNumbers(offset_dims=(),
                                   collapsed_slice_dims=(0,),
                                   start_index_map=(0,)),
        (1,), mode=lax.GatherScatterMode.PROMISE_IN_BOUNDS)


def _make_spmm(d, ch, nbuf):
    n_vreg = d // 16
    nr = ch // 128
    pr = 4 * nr
    nchunk = EPS_W // ch

    @functools.partial(
        pl.kernel,
        out_type=[jax.ShapeDtypeStruct((NP, d), jnp.float32),
                  jax.ShapeDtypeStruct((NP, d), jnp.float32)],
        mesh=_mesh,
        compiler_params=_sc_params,
        scratch_types=[
            pltpu.VMEM((nbuf * 4 * nr, 128), jnp.int32),
            pltpu.VMEM((nbuf * ch, d), jnp.float32),
            pltpu.VMEM_SHARED((NP, d), jnp.float32),
            pltpu.SemaphoreType.DMA,
            pltpu.SemaphoreType.DMA,
        ],
    )
    def spmm(msg_m, msg_v, ed_hbm, out_m, out_v,
             ed_v, rows_v, acc_sh, gsem, ssem):
        cid = lax.axis_index("c")
        sid = lax.axis_index("s")

        zval = jnp.zeros((16,), jnp.float32)

        def zbody(i, _):
            for k in range(n_vreg):
                rows_v[i, pl.ds(k * 16, 16)] = zval
            return 0

        lax.fori_loop(0, 128, zbody, 0)
        for z in range(RSUB // 128):
            pltpu.sync_copy(rows_v.at[pl.ds(0, 128)],
                            acc_sh.at[pl.ds(sid * RSUB + z * 128, 128)])
        plsc.subcore_barrier()

        bbase = sid * nchunk

        def issue_gather(p):
            for j in range(nr):
                dst = rows_v.at[pl.ds(p * ch + j * 128, 128)]

                @pl.when(cid == 0)
                def _():
                    pltpu.async_copy(msg_m.at[ed_v.at[p * pr + j]], dst, gsem)

                @pl.when(cid == 1)
                def _():
                    pltpu.async_copy(msg_v.at[ed_v.at[p * pr + j]], dst, gsem)

        def wait_gather(p):
            for j in range(nr):
                pltpu.make_async_copy(
                    msg_m.at[ed_v.at[p * pr + j]],
                    rows_v.at[pl.ds(p * ch + j * 128, 128)], gsem).wait()

        def issue_scatter(p):
            for j in range(nr):
                pltpu.async_copy(
                    rows_v.at[pl.ds(p * ch + j * 128, 128)],
                    acc_sh.at[ed_v.at[p * pr + nr + j]], ssem, add=True)

        def wait_scatter(p):
            for j in range(nr):
                pltpu.make_async_copy(
                    rows_v.at[pl.ds(p * ch + j * 128, 128)],
                    acc_sh.at[ed_v.at[p * pr + nr + j]], ssem).wait()

        pltpu.sync_copy(ed_hbm.at[bbase], ed_v.at[pl.ds(0, pr)])
        issue_gather(jnp.int32(0))

        def chunk_body(ci, _):
            p = lax.rem(ci, nbuf)
            q = lax.rem(ci + 1, nbuf)
            wait_gather(p)

            @pl.when(ci >= nbuf - 1)
            def _():
                wait_scatter(q)

            @pl.when(ci + 1 < nchunk)
            def _():
                pltpu.sync_copy(ed_hbm.at[bbase + ci + 1],
                                ed_v.at[pl.ds(q * pr, pr)])
                issue_gather(q)

            wrow0 = p * pr + 2 * nr + cid * nr

            def grp(g, _):
                j = lax.div(g, 8)
                col = lax.rem(g, 8) * 16
                wi = ed_v[wrow0 + j, pl.ds(col, 16)]
                wv = plsc.bitcast(wi, jnp.float32)
                ebase = p * ch + g * 16
                for u in range(16):
                    ws = _lane_splat(wv, u)
                    e = ebase + u
                    for kk in range(n_vreg):
                        rows_v[e, pl.ds(kk * 16, 16)] = (
                            rows_v[e, pl.ds(kk * 16, 16)] * ws)
                return 0

            lax.fori_loop(0, ch // 16, grp, 0)
            issue_scatter(p)
            return 0

        lax.fori_loop(0, nchunk, chunk_body, 0)
        for c in range(max(0, nchunk - nbuf + 1), nchunk):
            wait_scatter(jnp.int32(c % nbuf))
        plsc.subcore_barrier()

        sl = pl.ds(sid * RSUB, RSUB)

        @pl.when(cid == 0)
        def _():
            pltpu.sync_copy(acc_sh.at[sl], out_m.at[sl])

        @pl.when(cid == 1)
        def _():
            pltpu.sync_copy(acc_sh.at[sl], out_v.at[sl])

    return spmm


CH1, NBUF1 = 256, 2
CH2, NBUF2 = 512, 3
_spmm1 = _make_spmm(D_HID, CH1, NBUF1)
_spmm2 = _make_spmm(16, CH2, NBUF2)


@functools.partial(
    pl.kernel,
    out_type=jax.ShapeDtypeStruct((BP, 16), jnp.float32),
    mesh=_mesh,
    compiler_params=_sc_params,
    scratch_types=[
        pltpu.VMEM((BP // NW,), jnp.int32),
        pltpu.VMEM((BP // NW, 16), jnp.float32),
        pltpu.SemaphoreType.DMA,
    ],
)
def _gather_out(hs_hbm, idx_hbm, out_hbm, idx_v, rows_v, sem):
    wid = lax.axis_index("s") * 2 + lax.axis_index("c")
    b_per_w = BP // NW
    base = wid * b_per_w
    pltpu.sync_copy(idx_hbm.at[pl.ds(base, b_per_w)], idx_v)
    pltpu.async_copy(hs_hbm.at[idx_v], rows_v, sem).wait()
    pltpu.sync_copy(rows_v, out_hbm.at[pl.ds(base, b_per_w)])


def kernel(x, edge_index, adj1_values, adj2_values, node_index,
           kernel_f, kernel_mean, kernel_var):
    src = edge_index[0]
    dst = edge_index[1]
    epad = EP - E
    src_p = jnp.concatenate([src, jnp.zeros((epad,), jnp.int32)])
    dst_p = jnp.concatenate([dst, jnp.full((epad,), N, jnp.int32)])
    w1_p = jnp.concatenate([adj1_values, jnp.zeros((epad,), jnp.float32)])
    w2_p = jnp.concatenate([adj2_values, jnp.zeros((epad,), jnp.float32)])

    w1i = lax.bitcast_convert_type(w1_p, jnp.int32)
    w2i = lax.bitcast_convert_type(w2_p, jnp.int32)

    def _pack(ch):
        nr = ch // 128
        nchunk = EPS_W // ch

        def r(a):
            return a.reshape(16, nchunk, nr, 128)

        return jnp.concatenate([r(src_p), r(dst_p), r(w1i), r(w2i)],
                               axis=2).reshape(16 * nchunk, 4 * nr, 128)

    ed1 = _pack(CH1)
    ed2 = _pack(CH2)
    xp = jnp.pad(x, ((0, NP - N), (0, 0)))
    kmp = jnp.pad(kernel_mean, ((0, 0), (0, 16 - D_OUT)))
    kvp = jnp.pad(kernel_var, ((0, 0), (0, 16 - D_OUT)))
    eps = jax.random.normal(jax.random.key(42), (N, D_OUT), dtype=jnp.float32)
    epsp = jnp.pad(eps, ((0, NP - N), (0, 16 - D_OUT)))
    ni_p = jnp.pad(node_index, (0, BP - B))

    msg_m, msg_v = _tc_a(xp, kernel_f)
    am, av = _spmm1(msg_m, msg_v, ed1)
    m2, v2 = _tc_b(am, av, kmp, kvp)
    qm, qv = _spmm2(m2, v2, ed2)
    hs = _tc_d(qm, qv, epsp)
    out = _gather_out(hs, ni_p)
    return out[:B, :D_OUT]


# speedup vs baseline: 9.0919x; 1.2943x over previous
import functools

import jax
import jax.numpy as jnp
from jax import lax
from jax.experimental import pallas as pl
from jax.experimental.pallas import tpu as pltpu
from jax.experimental.pallas import tpu_sc as plsc

N = 10000
E = 320000
D_IN = 128
D_HID = 64
D_OUT = 7
B = 1000
GAMMA = 1.0

NP = 10240
EPS_W = 20480
EP = EPS_W * 16
RSUB = NP // 16
BP = 1024
NW = 32

_mesh = plsc.VectorSubcoreMesh(
    core_axis_name="c", subcore_axis_name="s", num_cores=2, num_subcores=16)

_sc_params = pltpu.CompilerParams(
    needs_layout_passes=False, use_tc_tiling_on_sc=False)


def _tca_body(x_ref, kf_ref, om_ref, ov_ref):
    h = jnp.dot(x_ref[...], kf_ref[...], preferred_element_type=jnp.float32)
    m = jnp.maximum(h, 0.0)
    a = jnp.exp(-GAMMA * m)
    ma = m * a
    om_ref[...] = ma
    ov_ref[...] = ma * a


def _tc_a(xp, kf):
    return pl.pallas_call(
        _tca_body,
        grid=(NP // 256,),
        in_specs=[
            pl.BlockSpec((256, D_IN), lambda i: (i, 0)),
            pl.BlockSpec((D_IN, D_HID), lambda i: (0, 0)),
        ],
        out_specs=[pl.BlockSpec((256, D_HID), lambda i: (i, 0)),
                   pl.BlockSpec((256, D_HID), lambda i: (i, 0))],
        out_shape=[jax.ShapeDtypeStruct((NP, D_HID), jnp.float32),
                   jax.ShapeDtypeStruct((NP, D_HID), jnp.float32)],
    )(xp, kf)


def _tcb_body(am_ref, av_ref, km_ref, kv_ref, om_ref, ov_ref):
    m2 = jnp.dot(am_ref[...], km_ref[...], preferred_element_type=jnp.float32)
    v2 = jnp.maximum(
        jnp.dot(av_ref[...], kv_ref[...], preferred_element_type=jnp.float32),
        0.0)
    a = jnp.exp(-GAMMA * v2)
    om_ref[...] = m2 * a
    ov_ref[...] = v2 * a * a


def _tc_b(am, av, kmp, kvp):
    return pl.pallas_call(
        _tcb_body,
        grid=(NP // 256,),
        in_specs=[
            pl.BlockSpec((256, D_HID), lambda i: (i, 0)),
            pl.BlockSpec((256, D_HID), lambda i: (i, 0)),
            pl.BlockSpec((D_HID, 16), lambda i: (0, 0)),
            pl.BlockSpec((D_HID, 16), lambda i: (0, 0)),
        ],
        out_specs=[pl.BlockSpec((256, 16), lambda i: (i, 0)),
                   pl.BlockSpec((256, 16), lambda i: (i, 0))],
        out_shape=[jax.ShapeDtypeStruct((NP, 16), jnp.float32),
                   jax.ShapeDtypeStruct((NP, 16), jnp.float32)],
    )(am, av, kmp, kvp)


def _tcd_body(qm_ref, qv_ref, eps_ref, o_ref):
    o_ref[...] = qm_ref[...] + jnp.sqrt(qv_ref[...] + 1e-8) * eps_ref[...]


def _tc_d(qm, qv, epsp):
    return pl.pallas_call(
        _tcd_body,
        grid=(NP // 1024,),
        in_specs=[
            pl.BlockSpec((1024, 16), lambda i: (i, 0)),
            pl.BlockSpec((1024, 16), lambda i: (i, 0)),
            pl.BlockSpec((1024, 16), lambda i: (i, 0)),
        ],
        out_specs=pl.BlockSpec((1024, 16), lambda i: (i, 0)),
        out_shape=jax.ShapeDtypeStruct((NP, 16), jnp.float32),
    )(qm, qv, epsp)


def _lane_splat(vec, u):
    return lax.gather(
        vec, jnp.full((16, 1), u, jnp.int32),
        lax.GatherDimensionNumbers(offset_dims=(),
                                   collapsed_slice_dims=(0,),
                                   start_index_map=(0,)),
        (1,), mode=lax.GatherScatterMode.PROMISE_IN_BOUNDS)


def _make_spmm(d, ch, nbuf):
    n_vreg = d // 16
    nr = ch // 128
    pr = 4 * nr
    nchunk = EPS_W // ch

    @functools.partial(
        pl.kernel,
        out_type=[jax.ShapeDtypeStruct((NP, d), jnp.float32),
                  jax.ShapeDtypeStruct((NP, d), jnp.float32)],
        mesh=_mesh,
        compiler_params=_sc_params,
        scratch_types=[
            pltpu.VMEM((nbuf * 4 * nr, 128), jnp.int32),
            pltpu.VMEM((nbuf * ch, d), jnp.float32),
            pltpu.VMEM_SHARED((NP, d), jnp.float32),
            pltpu.SemaphoreType.DMA,
            pltpu.SemaphoreType.DMA,
        ],
    )
    def spmm(msg_m, msg_v, ed_hbm, out_m, out_v,
             ed_v, rows_v, acc_sh, gsem, ssem):
        cid = lax.axis_index("c")
        sid = lax.axis_index("s")

        zval = jnp.zeros((16,), jnp.float32)

        def zbody(i, _):
            for k in range(n_vreg):
                rows_v[i, pl.ds(k * 16, 16)] = zval
            return 0

        lax.fori_loop(0, 128, zbody, 0)
        for z in range(RSUB // 128):
            pltpu.sync_copy(rows_v.at[pl.ds(0, 128)],
                            acc_sh.at[pl.ds(sid * RSUB + z * 128, 128)])
        plsc.subcore_barrier()

        bbase = sid * nchunk

        def issue_gather(p):
            for j in range(nr):
                dst = rows_v.at[pl.ds(p * ch + j * 128, 128)]

                @pl.when(cid == 0)
                def _():
                    pltpu.async_copy(msg_m.at[ed_v.at[p * pr + j]], dst, gsem)

                @pl.when(cid == 1)
                def _():
                    pltpu.async_copy(msg_v.at[ed_v.at[p * pr + j]], dst, gsem)

        def wait_gather(p):
            for j in range(nr):
                pltpu.make_async_copy(
                    msg_m.at[ed_v.at[p * pr + j]],
                    rows_v.at[pl.ds(p * ch + j * 128, 128)], gsem).wait()

        def issue_scatter(p):
            for j in range(nr):
                pltpu.async_copy(
                    rows_v.at[pl.ds(p * ch + j * 128, 128)],
                    acc_sh.at[ed_v.at[p * pr + nr + j]], ssem, add=True)

        def wait_scatter(p):
            for j in range(nr):
                pltpu.make_async_copy(
                    rows_v.at[pl.ds(p * ch + j * 128, 128)],
                    acc_sh.at[ed_v.at[p * pr + nr + j]], ssem).wait()

        pltpu.sync_copy(ed_hbm.at[bbase], ed_v.at[pl.ds(0, pr)])
        issue_gather(jnp.int32(0))

        def chunk_body(ci, _):
            p = lax.rem(ci, nbuf)
            q = lax.rem(ci + 1, nbuf)
            wait_gather(p)

            @pl.when(ci >= nbuf - 1)
            def _():
                wait_scatter(q)

            @pl.when(ci + 1 < nchunk)
            def _():
                pltpu.sync_copy(ed_hbm.at[bbase + ci + 1],
                                ed_v.at[pl.ds(q * pr, pr)])
                issue_gather(q)

            wrow0 = p * pr + 2 * nr + cid * nr

            @plsc.parallel_loop(0, ch // 16, unroll=2)
            def grp(g):
                j = lax.div(g, 8)
                col = lax.rem(g, 8) * 16
                wi = ed_v[wrow0 + j, pl.ds(col, 16)]
                wv = plsc.bitcast(wi, jnp.float32)
                ebase = p * ch + g * 16
                for u in range(16):
                    ws = _lane_splat(wv, u)
                    e = ebase + u
                    for kk in range(n_vreg):
                        rows_v[e, pl.ds(kk * 16, 16)] = (
                            rows_v[e, pl.ds(kk * 16, 16)] * ws)

            issue_scatter(p)
            return 0

        lax.fori_loop(0, nchunk, chunk_body, 0)
        for c in range(max(0, nchunk - nbuf + 1), nchunk):
            wait_scatter(jnp.int32(c % nbuf))
        plsc.subcore_barrier()

        sl = pl.ds(sid * RSUB, RSUB)

        @pl.when(cid == 0)
        def _():
            pltpu.sync_copy(acc_sh.at[sl], out_m.at[sl])

        @pl.when(cid == 1)
        def _():
            pltpu.sync_copy(acc_sh.at[sl], out_v.at[sl])

    return spmm


CH1, NBUF1 = 256, 2
CH2, NBUF2 = 512, 3
_spmm1 = _make_spmm(D_HID, CH1, NBUF1)
_spmm2 = _make_spmm(16, CH2, NBUF2)


@functools.partial(
    pl.kernel,
    out_type=jax.ShapeDtypeStruct((BP, 16), jnp.float32),
    mesh=_mesh,
    compiler_params=_sc_params,
    scratch_types=[
        pltpu.VMEM((BP // NW,), jnp.int32),
        pltpu.VMEM((BP // NW, 16), jnp.float32),
        pltpu.SemaphoreType.DMA,
    ],
)
def _gather_out(hs_hbm, idx_hbm, out_hbm, idx_v, rows_v, sem):
    wid = lax.axis_index("s") * 2 + lax.axis_index("c")
    b_per_w = BP // NW
    base = wid * b_per_w
    pltpu.sync_copy(idx_hbm.at[pl.ds(base, b_per_w)], idx_v)
    pltpu.async_copy(hs_hbm.at[idx_v], rows_v, sem).wait()
    pltpu.sync_copy(rows_v, out_hbm.at[pl.ds(base, b_per_w)])


def kernel(x, edge_index, adj1_values, adj2_values, node_index,
           kernel_f, kernel_mean, kernel_var):
    src = edge_index[0]
    dst = edge_index[1]
    epad = EP - E
    src_p = jnp.concatenate([src, jnp.zeros((epad,), jnp.int32)])
    dst_p = jnp.concatenate([dst, jnp.full((epad,), N, jnp.int32)])
    w1_p = jnp.concatenate([adj1_values, jnp.zeros((epad,), jnp.float32)])
    w2_p = jnp.concatenate([adj2_values, jnp.zeros((epad,), jnp.float32)])

    w1i = lax.bitcast_convert_type(w1_p, jnp.int32)
    w2i = lax.bitcast_convert_type(w2_p, jnp.int32)

    def _pack(ch):
        nr = ch // 128
        nchunk = EPS_W // ch

        def r(a):
            return a.reshape(16, nchunk, nr, 128)

        return jnp.concatenate([r(src_p), r(dst_p), r(w1i), r(w2i)],
                               axis=2).reshape(16 * nchunk, 4 * nr, 128)

    ed1 = _pack(CH1)
    ed2 = _pack(CH2)
    xp = jnp.pad(x, ((0, NP - N), (0, 0)))
    kmp = jnp.pad(kernel_mean, ((0, 0), (0, 16 - D_OUT)))
    kvp = jnp.pad(kernel_var, ((0, 0), (0, 16 - D_OUT)))
    eps = jax.random.normal(jax.random.key(42), (N, D_OUT), dtype=jnp.float32)
    epsp = jnp.pad(eps, ((0, NP - N), (0, 16 - D_OUT)))
    ni_p = jnp.pad(node_index, (0, BP - B))

    msg_m, msg_v = _tc_a(xp, kernel_f)
    am, av = _spmm1(msg_m, msg_v, ed1)
    m2, v2 = _tc_b(am, av, kmp, kvp)
    qm, qv = _spmm2(m2, v2, ed2)
    hs = _tc_d(qm, qv, epsp)
    out = _gather_out(hs, ni_p)
    return out[:B, :D_OUT]


# speedup vs baseline: 10.0131x; 1.1013x over previous
import functools

import jax
import jax.numpy as jnp
from jax import lax
from jax.experimental import pallas as pl
from jax.experimental.pallas import tpu as pltpu
from jax.experimental.pallas import tpu_sc as plsc

N = 10000
E = 320000
D_IN = 128
D_HID = 64
D_OUT = 7
B = 1000
GAMMA = 1.0

NP = 10240
EPS_W = 20480
EP = EPS_W * 16
RSUB = NP // 16
BP = 1024
NW = 32

_mesh = plsc.VectorSubcoreMesh(
    core_axis_name="c", subcore_axis_name="s", num_cores=2, num_subcores=16)

_sc_params = pltpu.CompilerParams(
    needs_layout_passes=False, use_tc_tiling_on_sc=False)


def _tca_body(x_ref, kf_ref, om_ref, ov_ref):
    h = jnp.dot(x_ref[...], kf_ref[...], preferred_element_type=jnp.float32)
    m = jnp.maximum(h, 0.0)
    a = jnp.exp(-GAMMA * m)
    ma = m * a
    om_ref[...] = ma
    ov_ref[...] = ma * a


def _tc_a(xp, kf):
    return pl.pallas_call(
        _tca_body,
        grid=(NP // 256,),
        in_specs=[
            pl.BlockSpec((256, D_IN), lambda i: (i, 0)),
            pl.BlockSpec((D_IN, D_HID), lambda i: (0, 0)),
        ],
        out_specs=[pl.BlockSpec((256, D_HID), lambda i: (i, 0)),
                   pl.BlockSpec((256, D_HID), lambda i: (i, 0))],
        out_shape=[jax.ShapeDtypeStruct((NP, D_HID), jnp.float32),
                   jax.ShapeDtypeStruct((NP, D_HID), jnp.float32)],
    )(xp, kf)


def _tcb_body(am_ref, av_ref, km_ref, kv_ref, om_ref, ov_ref):
    m2 = jnp.dot(am_ref[...], km_ref[...], preferred_element_type=jnp.float32)
    v2 = jnp.maximum(
        jnp.dot(av_ref[...], kv_ref[...], preferred_element_type=jnp.float32),
        0.0)
    a = jnp.exp(-GAMMA * v2)
    om_ref[...] = m2 * a
    ov_ref[...] = v2 * a * a


def _tc_b(am, av, kmp, kvp):
    return pl.pallas_call(
        _tcb_body,
        grid=(NP // 256,),
        in_specs=[
            pl.BlockSpec((256, D_HID), lambda i: (i, 0)),
            pl.BlockSpec((256, D_HID), lambda i: (i, 0)),
            pl.BlockSpec((D_HID, 16), lambda i: (0, 0)),
            pl.BlockSpec((D_HID, 16), lambda i: (0, 0)),
        ],
        out_specs=[pl.BlockSpec((256, 16), lambda i: (i, 0)),
                   pl.BlockSpec((256, 16), lambda i: (i, 0))],
        out_shape=[jax.ShapeDtypeStruct((NP, 16), jnp.float32),
                   jax.ShapeDtypeStruct((NP, 16), jnp.float32)],
    )(am, av, kmp, kvp)


def _tcd_body(qm_ref, qv_ref, eps_ref, o_ref):
    o_ref[...] = qm_ref[...] + jnp.sqrt(qv_ref[...] + 1e-8) * eps_ref[...]


def _tc_d(qm, qv, epsp):
    return pl.pallas_call(
        _tcd_body,
        grid=(NP // 1024,),
        in_specs=[
            pl.BlockSpec((1024, 16), lambda i: (i, 0)),
            pl.BlockSpec((1024, 16), lambda i: (i, 0)),
            pl.BlockSpec((1024, 16), lambda i: (i, 0)),
        ],
        out_specs=pl.BlockSpec((1024, 16), lambda i: (i, 0)),
        out_shape=jax.ShapeDtypeStruct((NP, 16), jnp.float32),
    )(qm, qv, epsp)


def _lane_splat(vec, u):
    return lax.gather(
        vec, jnp.full((16, 1), u, jnp.int32),
        lax.GatherDimensionNumbers(offset_dims=(),
                                   collapsed_slice_dims=(0,),
                                   start_index_map=(0,)),
        (1,), mode=lax.GatherScatterMode.PROMISE_IN_BOUNDS)


def _make_spmm(d, ch, nbuf):
    n_vreg = d // 16
    nr = ch // 128
    pr = 4 * nr
    nchunk = EPS_W // ch

    @functools.partial(
        pl.kernel,
        out_type=[jax.ShapeDtypeStruct((NP, d), jnp.float32),
                  jax.ShapeDtypeStruct((NP, d), jnp.float32)],
        mesh=_mesh,
        compiler_params=_sc_params,
        scratch_types=[
            pltpu.VMEM(((nbuf + 1) * 4 * nr, 128), jnp.int32),
            pltpu.VMEM((nbuf * ch, d), jnp.float32),
            pltpu.VMEM_SHARED((NP, d), jnp.float32),
            pltpu.SemaphoreType.DMA,
            pltpu.SemaphoreType.DMA,
            pltpu.SemaphoreType.DMA,
        ],
    )
    def spmm(msg_m, msg_v, ed_hbm, out_m, out_v,
             ed_v, rows_v, acc_sh, gsem, ssem, esem):
        cid = lax.axis_index("c")
        sid = lax.axis_index("s")

        zval = jnp.zeros((16,), jnp.float32)

        def zbody(i, _):
            for k in range(n_vreg):
                rows_v[i, pl.ds(k * 16, 16)] = zval
            return 0

        lax.fori_loop(0, 128, zbody, 0)
        for z in range(RSUB // 128):
            pltpu.sync_copy(rows_v.at[pl.ds(0, 128)],
                            acc_sh.at[pl.ds(sid * RSUB + z * 128, 128)])
        plsc.subcore_barrier()

        bbase = sid * nchunk
        nes = nbuf + 1

        def issue_gather(p, es):
            for j in range(nr):
                dst = rows_v.at[pl.ds(p * ch + j * 128, 128)]

                @pl.when(cid == 0)
                def _():
                    pltpu.async_copy(msg_m.at[ed_v.at[es * pr + j]], dst, gsem)

                @pl.when(cid == 1)
                def _():
                    pltpu.async_copy(msg_v.at[ed_v.at[es * pr + j]], dst, gsem)

        def wait_gather(p, es):
            for j in range(nr):
                pltpu.make_async_copy(
                    msg_m.at[ed_v.at[es * pr + j]],
                    rows_v.at[pl.ds(p * ch + j * 128, 128)], gsem).wait()

        def issue_scatter(p, es):
            for j in range(nr):
                pltpu.async_copy(
                    rows_v.at[pl.ds(p * ch + j * 128, 128)],
                    acc_sh.at[ed_v.at[es * pr + nr + j]], ssem, add=True)

        def wait_scatter(p, es):
            for j in range(nr):
                pltpu.make_async_copy(
                    rows_v.at[pl.ds(p * ch + j * 128, 128)],
                    acc_sh.at[ed_v.at[es * pr + nr + j]], ssem).wait()

        pltpu.sync_copy(ed_hbm.at[bbase], ed_v.at[pl.ds(0, pr)])
        issue_gather(jnp.int32(0), jnp.int32(0))
        if nchunk > 1:
            pltpu.async_copy(ed_hbm.at[bbase + 1],
                             ed_v.at[pl.ds(pr, pr)], esem)

        def chunk_body(ci, _):
            p = lax.rem(ci, nbuf)
            q = lax.rem(ci + 1, nbuf)
            es = lax.rem(ci, nes)
            es1 = lax.rem(ci + 1, nes)
            es2 = lax.rem(ci + 2, nes)
            wait_gather(p, es)

            @pl.when(ci >= nbuf - 1)
            def _():
                wait_scatter(q, lax.rem(ci + 1 - nbuf, nes))

            @pl.when(ci + 1 < nchunk)
            def _():
                pltpu.make_async_copy(ed_hbm.at[bbase],
                                      ed_v.at[pl.ds(es1 * pr, pr)],
                                      esem).wait()
                issue_gather(q, es1)

            @pl.when(ci + 2 < nchunk)
            def _():
                pltpu.async_copy(ed_hbm.at[bbase + ci + 2],
                                 ed_v.at[pl.ds(es2 * pr, pr)], esem)

            wrow0 = es * pr + 2 * nr + cid * nr

            @plsc.parallel_loop(0, ch // 16, unroll=2)
            def grp(g):
                j = lax.div(g, 8)
                col = lax.rem(g, 8) * 16
                wi = ed_v[wrow0 + j, pl.ds(col, 16)]
                wv = plsc.bitcast(wi, jnp.float32)
                ebase = p * ch + g * 16
                for u in range(16):
                    ws = _lane_splat(wv, u)
                    e = ebase + u
                    for kk in range(n_vreg):
                        rows_v[e, pl.ds(kk * 16, 16)] = (
                            rows_v[e, pl.ds(kk * 16, 16)] * ws)

            issue_scatter(p, es)
            return 0

        lax.fori_loop(0, nchunk, chunk_body, 0)
        for c in range(max(0, nchunk - nbuf + 1), nchunk):
            wait_scatter(jnp.int32(c % nbuf), jnp.int32(c % nes))
        plsc.subcore_barrier()

        sl = pl.ds(sid * RSUB, RSUB)

        @pl.when(cid == 0)
        def _():
            pltpu.sync_copy(acc_sh.at[sl], out_m.at[sl])

        @pl.when(cid == 1)
        def _():
            pltpu.sync_copy(acc_sh.at[sl], out_v.at[sl])

    return spmm


CH1, NBUF1 = 256, 2
CH2, NBUF2 = 512, 3
_spmm1 = _make_spmm(D_HID, CH1, NBUF1)
_spmm2 = _make_spmm(16, CH2, NBUF2)


@functools.partial(
    pl.kernel,
    out_type=jax.ShapeDtypeStruct((BP, 16), jnp.float32),
    mesh=_mesh,
    compiler_params=_sc_params,
    scratch_types=[
        pltpu.VMEM((BP // NW,), jnp.int32),
        pltpu.VMEM((BP // NW, 16), jnp.float32),
        pltpu.SemaphoreType.DMA,
    ],
)
def _gather_out(hs_hbm, idx_hbm, out_hbm, idx_v, rows_v, sem):
    wid = lax.axis_index("s") * 2 + lax.axis_index("c")
    b_per_w = BP // NW
    base = wid * b_per_w
    pltpu.sync_copy(idx_hbm.at[pl.ds(base, b_per_w)], idx_v)
    pltpu.async_copy(hs_hbm.at[idx_v], rows_v, sem).wait()
    pltpu.sync_copy(rows_v, out_hbm.at[pl.ds(base, b_per_w)])


def kernel(x, edge_index, adj1_values, adj2_values, node_index,
           kernel_f, kernel_mean, kernel_var):
    src = edge_index[0]
    dst = edge_index[1]
    epad = EP - E
    src_p = jnp.concatenate([src, jnp.zeros((epad,), jnp.int32)])
    dst_p = jnp.concatenate([dst, jnp.full((epad,), N, jnp.int32)])
    w1_p = jnp.concatenate([adj1_values, jnp.zeros((epad,), jnp.float32)])
    w2_p = jnp.concatenate([adj2_values, jnp.zeros((epad,), jnp.float32)])

    w1i = lax.bitcast_convert_type(w1_p, jnp.int32)
    w2i = lax.bitcast_convert_type(w2_p, jnp.int32)

    def _pack(ch):
        nr = ch // 128
        nchunk = EPS_W // ch

        def r(a):
            return a.reshape(16, nchunk, nr, 128)

        return jnp.concatenate([r(src_p), r(dst_p), r(w1i), r(w2i)],
                               axis=2).reshape(16 * nchunk, 4 * nr, 128)

    ed1 = _pack(CH1)
    ed2 = _pack(CH2)
    xp = jnp.pad(x, ((0, NP - N), (0, 0)))
    kmp = jnp.pad(kernel_mean, ((0, 0), (0, 16 - D_OUT)))
    kvp = jnp.pad(kernel_var, ((0, 0), (0, 16 - D_OUT)))
    eps = jax.random.normal(jax.random.key(42), (N, D_OUT), dtype=jnp.float32)
    epsp = jnp.pad(eps, ((0, NP - N), (0, 16 - D_OUT)))
    ni_p = jnp.pad(node_index, (0, BP - B))

    msg_m, msg_v = _tc_a(xp, kernel_f)
    am, av = _spmm1(msg_m, msg_v, ed1)
    m2, v2 = _tc_b(am, av, kmp, kvp)
    qm, qv = _spmm2(m2, v2, ed2)
    hs = _tc_d(qm, qv, epsp)
    out = _gather_out(hs, ni_p)
    return out[:B, :D_OUT]


# speedup vs baseline: 10.0142x; 1.0001x over previous
import functools

import jax
import jax.numpy as jnp
from jax import lax
from jax.experimental import pallas as pl
from jax.experimental.pallas import tpu as pltpu
from jax.experimental.pallas import tpu_sc as plsc

N = 10000
E = 320000
D_IN = 128
D_HID = 64
D_OUT = 7
B = 1000
GAMMA = 1.0

NP = 10240
EPS_W = 20480
EP = EPS_W * 16
RSUB = NP // 16
BP = 1024
NW = 32

_mesh = plsc.VectorSubcoreMesh(
    core_axis_name="c", subcore_axis_name="s", num_cores=2, num_subcores=16)

_sc_params = pltpu.CompilerParams(
    needs_layout_passes=False, use_tc_tiling_on_sc=False)


def _tca_body(x_ref, kf_ref, om_ref, ov_ref):
    h = jnp.dot(x_ref[...], kf_ref[...], preferred_element_type=jnp.float32)
    m = jnp.maximum(h, 0.0)
    a = jnp.exp(-GAMMA * m)
    ma = m * a
    om_ref[...] = ma
    ov_ref[...] = ma * a


def _tc_a(xp, kf):
    return pl.pallas_call(
        _tca_body,
        grid=(NP // 256,),
        in_specs=[
            pl.BlockSpec((256, D_IN), lambda i: (i, 0)),
            pl.BlockSpec((D_IN, D_HID), lambda i: (0, 0)),
        ],
        out_specs=[pl.BlockSpec((256, D_HID), lambda i: (i, 0)),
                   pl.BlockSpec((256, D_HID), lambda i: (i, 0))],
        out_shape=[jax.ShapeDtypeStruct((NP, D_HID), jnp.float32),
                   jax.ShapeDtypeStruct((NP, D_HID), jnp.float32)],
    )(xp, kf)


def _tcb_body(am_ref, av_ref, km_ref, kv_ref, om_ref, ov_ref):
    m2 = jnp.dot(am_ref[...], km_ref[...], preferred_element_type=jnp.float32)
    v2 = jnp.maximum(
        jnp.dot(av_ref[...], kv_ref[...], preferred_element_type=jnp.float32),
        0.0)
    a = jnp.exp(-GAMMA * v2)
    om_ref[...] = m2 * a
    ov_ref[...] = v2 * a * a


def _tc_b(am, av, kmp, kvp):
    return pl.pallas_call(
        _tcb_body,
        grid=(NP // 256,),
        in_specs=[
            pl.BlockSpec((256, D_HID), lambda i: (i, 0)),
            pl.BlockSpec((256, D_HID), lambda i: (i, 0)),
            pl.BlockSpec((D_HID, 16), lambda i: (0, 0)),
            pl.BlockSpec((D_HID, 16), lambda i: (0, 0)),
        ],
        out_specs=[pl.BlockSpec((256, 16), lambda i: (i, 0)),
                   pl.BlockSpec((256, 16), lambda i: (i, 0))],
        out_shape=[jax.ShapeDtypeStruct((NP, 16), jnp.float32),
                   jax.ShapeDtypeStruct((NP, 16), jnp.float32)],
    )(am, av, kmp, kvp)


def _tcd_body(qm_ref, qv_ref, eps_ref, o_ref):
    o_ref[...] = qm_ref[...] + jnp.sqrt(qv_ref[...] + 1e-8) * eps_ref[...]


def _tc_d(qm, qv, epsp):
    return pl.pallas_call(
        _tcd_body,
        grid=(NP // 1024,),
        in_specs=[
            pl.BlockSpec((1024, 16), lambda i: (i, 0)),
            pl.BlockSpec((1024, 16), lambda i: (i, 0)),
            pl.BlockSpec((1024, 16), lambda i: (i, 0)),
        ],
        out_specs=pl.BlockSpec((1024, 16), lambda i: (i, 0)),
        out_shape=jax.ShapeDtypeStruct((NP, 16), jnp.float32),
    )(qm, qv, epsp)


def _lane_splat(vec, u):
    return lax.gather(
        vec, jnp.full((16, 1), u, jnp.int32),
        lax.GatherDimensionNumbers(offset_dims=(),
                                   collapsed_slice_dims=(0,),
                                   start_index_map=(0,)),
        (1,), mode=lax.GatherScatterMode.PROMISE_IN_BOUNDS)


def _make_spmm(d, ch, nbuf):
    n_vreg = d // 16
    nr = ch // 128
    pr = 4 * nr
    nchunk = EPS_W // ch

    @functools.partial(
        pl.kernel,
        out_type=[jax.ShapeDtypeStruct((NP, d), jnp.float32),
                  jax.ShapeDtypeStruct((NP, d), jnp.float32)],
        mesh=_mesh,
        compiler_params=_sc_params,
        scratch_types=[
            pltpu.VMEM(((nbuf + 1) * 4 * nr, 128), jnp.int32),
            pltpu.VMEM((nbuf * ch, d), jnp.float32),
            pltpu.VMEM_SHARED((NP, d), jnp.float32),
            pltpu.SemaphoreType.DMA,
            pltpu.SemaphoreType.DMA,
            pltpu.SemaphoreType.DMA,
        ],
    )
    def spmm(msg_m, msg_v, ed_hbm, out_m, out_v,
             ed_v, rows_v, acc_sh, gsem, ssem, esem):
        cid = lax.axis_index("c")
        sid = lax.axis_index("s")

        zval = jnp.zeros((16,), jnp.float32)

        def zbody(i, _):
            for k in range(n_vreg):
                rows_v[i, pl.ds(k * 16, 16)] = zval
            return 0

        lax.fori_loop(0, 128, zbody, 0)
        for z in range(RSUB // 128):
            pltpu.sync_copy(rows_v.at[pl.ds(0, 128)],
                            acc_sh.at[pl.ds(sid * RSUB + z * 128, 128)])
        plsc.subcore_barrier()

        bbase = sid * nchunk
        nes = nbuf + 1

        def issue_gather(p, es):
            for j in range(nr):
                dst = rows_v.at[pl.ds(p * ch + j * 128, 128)]

                @pl.when(cid == 0)
                def _():
                    pltpu.async_copy(msg_m.at[ed_v.at[es * pr + j]], dst, gsem)

                @pl.when(cid == 1)
                def _():
                    pltpu.async_copy(msg_v.at[ed_v.at[es * pr + j]], dst, gsem)

        def wait_gather(p, es):
            for j in range(nr):
                pltpu.make_async_copy(
                    msg_m.at[ed_v.at[es * pr + j]],
                    rows_v.at[pl.ds(p * ch + j * 128, 128)], gsem).wait()

        def issue_scatter(p, es):
            for j in range(nr):
                pltpu.async_copy(
                    rows_v.at[pl.ds(p * ch + j * 128, 128)],
                    acc_sh.at[ed_v.at[es * pr + nr + j]], ssem, add=True)

        def wait_scatter(p, es):
            for j in range(nr):
                pltpu.make_async_copy(
                    rows_v.at[pl.ds(p * ch + j * 128, 128)],
                    acc_sh.at[ed_v.at[es * pr + nr + j]], ssem).wait()

        pltpu.sync_copy(ed_hbm.at[bbase], ed_v.at[pl.ds(0, pr)])
        issue_gather(jnp.int32(0), jnp.int32(0))
        if nchunk > 1:
            pltpu.async_copy(ed_hbm.at[bbase + 1],
                             ed_v.at[pl.ds(pr, pr)], esem)

        def chunk_body(ci, _):
            p = lax.rem(ci, nbuf)
            q = lax.rem(ci + 1, nbuf)
            es = lax.rem(ci, nes)
            es1 = lax.rem(ci + 1, nes)
            es2 = lax.rem(ci + 2, nes)
            wait_gather(p, es)

            @pl.when(ci >= nbuf - 1)
            def _():
                wait_scatter(q, lax.rem(ci + 1 - nbuf, nes))

            @pl.when(ci + 1 < nchunk)
            def _():
                pltpu.make_async_copy(ed_hbm.at[bbase],
                                      ed_v.at[pl.ds(es1 * pr, pr)],
                                      esem).wait()
                issue_gather(q, es1)

            @pl.when(ci + 2 < nchunk)
            def _():
                pltpu.async_copy(ed_hbm.at[bbase + ci + 2],
                                 ed_v.at[pl.ds(es2 * pr, pr)], esem)

            wrow0 = es * pr + 2 * nr + cid * nr

            @plsc.parallel_loop(0, ch // 16, unroll=4)
            def grp(g):
                j = lax.div(g, 8)
                col = lax.rem(g, 8) * 16
                wi = ed_v[wrow0 + j, pl.ds(col, 16)]
                wv = plsc.bitcast(wi, jnp.float32)
                ebase = p * ch + g * 16
                for u in range(16):
                    ws = _lane_splat(wv, u)
                    e = ebase + u
                    for kk in range(n_vreg):
                        rows_v[e, pl.ds(kk * 16, 16)] = (
                            rows_v[e, pl.ds(kk * 16, 16)] * ws)

            issue_scatter(p, es)
            return 0

        lax.fori_loop(0, nchunk, chunk_body, 0)
        for c in range(max(0, nchunk - nbuf + 1), nchunk):
            wait_scatter(jnp.int32(c % nbuf), jnp.int32(c % nes))
        plsc.subcore_barrier()

        sl = pl.ds(sid * RSUB, RSUB)

        @pl.when(cid == 0)
        def _():
            pltpu.sync_copy(acc_sh.at[sl], out_m.at[sl])

        @pl.when(cid == 1)
        def _():
            pltpu.sync_copy(acc_sh.at[sl], out_v.at[sl])

    return spmm


CH1, NBUF1 = 256, 2
CH2, NBUF2 = 512, 3
_spmm1 = _make_spmm(D_HID, CH1, NBUF1)
_spmm2 = _make_spmm(16, CH2, NBUF2)


@functools.partial(
    pl.kernel,
    out_type=jax.ShapeDtypeStruct((BP, 16), jnp.float32),
    mesh=_mesh,
    compiler_params=_sc_params,
    scratch_types=[
        pltpu.VMEM((BP // NW,), jnp.int32),
        pltpu.VMEM((BP // NW, 16), jnp.float32),
        pltpu.SemaphoreType.DMA,
    ],
)
def _gather_out(hs_hbm, idx_hbm, out_hbm, idx_v, rows_v, sem):
    wid = lax.axis_index("s") * 2 + lax.axis_index("c")
    b_per_w = BP // NW
    base = wid * b_per_w
    pltpu.sync_copy(idx_hbm.at[pl.ds(base, b_per_w)], idx_v)
    pltpu.async_copy(hs_hbm.at[idx_v], rows_v, sem).wait()
    pltpu.sync_copy(rows_v, out_hbm.at[pl.ds(base, b_per_w)])


def kernel(x, edge_index, adj1_values, adj2_values, node_index,
           kernel_f, kernel_mean, kernel_var):
    src = edge_index[0]
    dst = edge_index[1]
    epad = EP - E
    src_p = jnp.concatenate([src, jnp.zeros((epad,), jnp.int32)])
    dst_p = jnp.concatenate([dst, jnp.full((epad,), N, jnp.int32)])
    w1_p = jnp.concatenate([adj1_values, jnp.zeros((epad,), jnp.float32)])
    w2_p = jnp.concatenate([adj2_values, jnp.zeros((epad,), jnp.float32)])

    w1i = lax.bitcast_convert_type(w1_p, jnp.int32)
    w2i = lax.bitcast_convert_type(w2_p, jnp.int32)

    def _pack(ch):
        nr = ch // 128
        nchunk = EPS_W // ch

        def r(a):
            return a.reshape(16, nchunk, nr, 128)

        return jnp.concatenate([r(src_p), r(dst_p), r(w1i), r(w2i)],
                               axis=2).reshape(16 * nchunk, 4 * nr, 128)

    ed1 = _pack(CH1)
    ed2 = _pack(CH2)
    xp = jnp.pad(x, ((0, NP - N), (0, 0)))
    kmp = jnp.pad(kernel_mean, ((0, 0), (0, 16 - D_OUT)))
    kvp = jnp.pad(kernel_var, ((0, 0), (0, 16 - D_OUT)))
    eps = jax.random.normal(jax.random.key(42), (N, D_OUT), dtype=jnp.float32)
    epsp = jnp.pad(eps, ((0, NP - N), (0, 16 - D_OUT)))
    ni_p = jnp.pad(node_index, (0, BP - B))

    msg_m, msg_v = _tc_a(xp, kernel_f)
    am, av = _spmm1(msg_m, msg_v, ed1)
    m2, v2 = _tc_b(am, av, kmp, kvp)
    qm, qv = _spmm2(m2, v2, ed2)
    hs = _tc_d(qm, qv, epsp)
    out = _gather_out(hs, ni_p)
    return out[:B, :D_OUT]
